# baseline (speedup 1.0000x reference)
import jax
import jax.numpy as jnp
from jax import lax
from jax.experimental import pallas as pl
from jax.experimental.pallas import tpu as pltpu
from jax.experimental.pallas import tpu_sc as plsc

_BATCH = 32768
_NIN = 64
_NOUT = 256
_NC = 2
_NS = 16
_NW = _NC * _NS
_ROWS_PER_W = _BATCH // _NW
_CHUNK = 128
_NCHUNKS = _ROWS_PER_W // _CHUNK
_L = 16
_G = _NIN // _L
_U = 4


def _sc_body(xr_hbm, w_hbm, idx_hbm, out_hbm,
             x_v0, x_v1, out_v0, out_v1, w_v, idx_v,
             sem_x0, sem_x1, sem_o0, sem_o1):
    wid = lax.axis_index("s") * _NC + lax.axis_index("c")
    base_row = wid * _ROWS_PER_W

    x_bufs = (x_v0, x_v1)
    out_bufs = (out_v0, out_v1)
    x_sems = (sem_x0, sem_x1)
    o_sems = (sem_o0, sem_o1)

    x_dma0 = pltpu.async_copy(
        xr_hbm.at[pl.ds(pl.multiple_of(base_row // 2, 64), _CHUNK // 2)],
        x_v0, sem_x0)

    pltpu.sync_copy(w_hbm, w_v)
    pltpu.sync_copy(idx_hbm, idx_v)

    zero = jnp.zeros((_L,), jnp.float32)

    def zero_body(i, carry):
        r = i // (_NOUT // _L)
        k = (i % (_NOUT // _L)) * _L
        for b in range(2):
            out_bufs[b][r, pl.ds(k, _L)] = zero
            out_bufs[b][r + _CHUNK // 2, pl.ds(k, _L)] = zero
        return carry

    lax.fori_loop(0, _CHUNK // 2 * (_NOUT // _L), zero_body, 0,
                  unroll=4)

    w_regs = tuple(w_v[pl.ds(g * _L, _L)] for g in range(_G))
    idx_regs = tuple(idx_v[pl.ds(g * _L, _L)] for g in range(_G))

    x_dmas = [x_dma0, None]
    o_dmas = [None, None]
    for ci in range(_NCHUNKS):
        b = ci % 2
        row0 = base_row + ci * _CHUNK
        if ci + 1 < _NCHUNKS:
            nb = (ci + 1) % 2
            x_dmas[nb] = pltpu.async_copy(
                xr_hbm.at[pl.ds(pl.multiple_of((row0 + _CHUNK) // 2, 64),
                                _CHUNK // 2)],
                x_bufs[nb], x_sems[nb])
        x_dmas[b].wait()
        if o_dmas[b] is not None:
            o_dmas[b].wait()

        x_v = x_bufs[b]
        out_v = out_bufs[b]

        def row_body(i, carry):
            w_r, idx_r = carry
            for u in range(_U):
                r = i * _U + u
                vals = tuple(
                    x_v[r // 2, pl.ds((r % 2) * _NIN + g * _L, _L)] * w_r[g]
                    for g in range(_G))
                rsplat = jnp.full((_L,), r, jnp.int32)
                for g in range(_G):
                    plsc.store_scatter(out_v, [rsplat, idx_r[g]], vals[g])
            return carry

        lax.fori_loop(0, _CHUNK // _U, row_body, (w_regs, idx_regs))

        o_dmas[b] = pltpu.async_copy(
            out_v, out_hbm.at[pl.ds(row0, _CHUNK)], o_sems[b])

    for d in o_dmas:
        if d is not None:
            d.wait()


def kernel(X_in, weights, input_node_order):
    mesh = plsc.VectorSubcoreMesh(
        core_axis_name="c", subcore_axis_name="s",
        num_cores=_NC, num_subcores=_NS,
    )
    f = pl.kernel(
        _sc_body,
        out_type=jax.ShapeDtypeStruct((_BATCH, _NOUT), jnp.float32),
        mesh=mesh,
        compiler_params=pltpu.CompilerParams(needs_layout_passes=False),
        scratch_types=[
            pltpu.VMEM((_CHUNK // 2, 2 * _NIN), jnp.float32),
            pltpu.VMEM((_CHUNK // 2, 2 * _NIN), jnp.float32),
            pltpu.VMEM((_CHUNK, _NOUT), jnp.float32),
            pltpu.VMEM((_CHUNK, _NOUT), jnp.float32),
            pltpu.VMEM((_NIN,), jnp.float32),
            pltpu.VMEM((_NIN,), jnp.int32),
            pltpu.SemaphoreType.DMA,
            pltpu.SemaphoreType.DMA,
            pltpu.SemaphoreType.DMA,
            pltpu.SemaphoreType.DMA,
        ],
    )
    return f(X_in.reshape(_BATCH // 2, 2 * _NIN), weights, input_node_order)


# speedup vs baseline: 1.1978x; 1.1978x over previous
import jax
import jax.numpy as jnp
from jax import lax
from jax.experimental import pallas as pl
from jax.experimental.pallas import tpu as pltpu
from jax.experimental.pallas import tpu_sc as plsc

_BATCH = 32768
_NIN = 64
_NOUT = 256
_NC = 2
_NS = 16
_NW = _NC * _NS
_ROWS_PER_W = _BATCH // _NW
_CHUNK = 128
_NCHUNKS = _ROWS_PER_W // _CHUNK
_L = 16
_G = _NIN // _L
_U = 4


def _sc_body(x_hbm, w_hbm, idx_hbm, out_hbm,
             x_v0, x_v1, out_v0, out_v1, w_v, idx_v,
             sem_x0, sem_x1, sem_o0, sem_o1):
    wid = lax.axis_index("s") * _NC + lax.axis_index("c")
    base_row = wid * _ROWS_PER_W

    x_bufs = (x_v0, x_v1)
    out_bufs = (out_v0, out_v1)
    x_sems = (sem_x0, sem_x1)
    o_sems = (sem_o0, sem_o1)

    x_dma0 = pltpu.async_copy(x_hbm.at[pl.ds(base_row, _CHUNK)], x_v0, sem_x0)

    pltpu.sync_copy(w_hbm, w_v)
    pltpu.sync_copy(idx_hbm, idx_v)

    zero = jnp.zeros((_L,), jnp.float32)

    def zero_body(i, carry):
        r = i // (_NOUT // _L)
        k = (i % (_NOUT // _L)) * _L
        for b in range(2):
            out_bufs[b][r, pl.ds(k, _L)] = zero
            out_bufs[b][r + _CHUNK // 2, pl.ds(k, _L)] = zero
        return carry

    lax.fori_loop(0, _CHUNK // 2 * (_NOUT // _L), zero_body, 0,
                  unroll=4)

    w_regs = tuple(w_v[pl.ds(g * _L, _L)] for g in range(_G))
    idx_regs = tuple(idx_v[pl.ds(g * _L, _L)] for g in range(_G))

    x_dmas = [x_dma0, None]
    o_dmas = [None, None]
    for ci in range(_NCHUNKS):
        b = ci % 2
        row0 = base_row + ci * _CHUNK
        if ci + 1 < _NCHUNKS:
            nb = (ci + 1) % 2
            x_dmas[nb] = pltpu.async_copy(
                x_hbm.at[pl.ds(row0 + _CHUNK, _CHUNK)], x_bufs[nb], x_sems[nb])
        x_dmas[b].wait()
        if o_dmas[b] is not None:
            o_dmas[b].wait()

        x_v = x_bufs[b]
        out_v = out_bufs[b]

        def row_body(i, carry):
            w_r, idx_r = carry
            for u in range(_U):
                r = i * _U + u
                vals = tuple(x_v[r, pl.ds(g * _L, _L)] * w_r[g]
                             for g in range(_G))
                rsplat = jnp.full((_L,), r, jnp.int32)
                for g in range(_G):
                    plsc.store_scatter(out_v, [rsplat, idx_r[g]], vals[g])
            return carry

        lax.fori_loop(0, _CHUNK // _U, row_body, (w_regs, idx_regs))

        o_dmas[b] = pltpu.async_copy(
            out_v, out_hbm.at[pl.ds(row0, _CHUNK)], o_sems[b])

    for d in o_dmas:
        if d is not None:
            d.wait()


def kernel(X_in, weights, input_node_order):
    mesh = plsc.VectorSubcoreMesh(
        core_axis_name="c", subcore_axis_name="s",
        num_cores=_NC, num_subcores=_NS,
    )
    f = pl.kernel(
        _sc_body,
        out_type=jax.ShapeDtypeStruct((_BATCH, _NOUT), jnp.float32),
        mesh=mesh,
        compiler_params=pltpu.CompilerParams(needs_layout_passes=False),
        scratch_types=[
            pltpu.VMEM((_CHUNK, _NIN), jnp.float32),
            pltpu.VMEM((_CHUNK, _NIN), jnp.float32),
            pltpu.VMEM((_CHUNK, _NOUT), jnp.float32),
            pltpu.VMEM((_CHUNK, _NOUT), jnp.float32),
            pltpu.VMEM((_NIN,), jnp.float32),
            pltpu.VMEM((_NIN,), jnp.int32),
            pltpu.SemaphoreType.DMA,
            pltpu.SemaphoreType.DMA,
            pltpu.SemaphoreType.DMA,
            pltpu.SemaphoreType.DMA,
        ],
    )
    return f(X_in, weights, input_node_order)
